# baseline (speedup 1.0000x reference)
import functools

import jax
import jax.numpy as jnp
from jax import lax
from jax.experimental import pallas as pl
from jax.experimental.pallas import tpu as pltpu
from jax.experimental.pallas import tpu_sc as plsc

_NC = 2
_NS = 16
_NW = _NC * _NS

_CH_ROWS = 16


def _sc_add(in_hbm, pos_hbm, out_hbm, pos_v, buf_v, sd, bsd, chw, nch, b):
    cid = lax.axis_index("c")
    sid = lax.axis_index("s")
    wid = sid * _NC + cid
    base = wid * (nch * chw)

    def chunk_body(i, _):
        off = base + i * chw
        pltpu.sync_copy(pos_hbm.at[pl.ds(off, chw)], pos_v)

        def batch_body(bi, _):
            ioff = bi * sd + off
            pltpu.sync_copy(in_hbm.at[pl.ds(ioff, chw)], buf_v)

            @plsc.parallel_loop(0, chw, 16, unroll=8)
            def _add(j):
                buf_v[pl.ds(j, 16)] = buf_v[pl.ds(j, 16)] + pos_v[pl.ds(j, 16)]

            pltpu.sync_copy(buf_v, out_hbm.at[pl.ds(ioff, chw)])
            return 0

        lax.fori_loop(0, b, batch_body, 0)
        return 0

    lax.fori_loop(0, nch, chunk_body, 0)


def kernel(inputs, pos_table):
    B, S, D = inputs.shape
    CHW = _CH_ROWS * D
    NCH = S // (_NW * _CH_ROWS)
    SD = S * D

    mesh = plsc.VectorSubcoreMesh(core_axis_name="c", subcore_axis_name="s")
    body = functools.partial(_sc_add, sd=SD, bsd=B * SD, chw=CHW, nch=NCH, b=B)

    out_flat = pl.kernel(
        body,
        out_type=jax.ShapeDtypeStruct((B * SD,), jnp.float32),
        mesh=mesh,
        scratch_types=[
            pltpu.VMEM((CHW,), jnp.float32),
            pltpu.VMEM((CHW,), jnp.float32),
        ],
    )(inputs.reshape(B * SD), pos_table.reshape(SD))
    return out_flat.reshape(B, S, D)


# speedup vs baseline: 1.2451x; 1.2451x over previous
import functools

import jax
import jax.numpy as jnp
from jax import lax
from jax.experimental import pallas as pl
from jax.experimental.pallas import tpu as pltpu
from jax.experimental.pallas import tpu_sc as plsc

_NC = 2
_NS = 16
_NW = _NC * _NS

_CH = 8


def _sc_add(in_hbm, pos_hbm, out_hbm,
            in0, in1, ou0, ou1, ps0, ps1,
            is0, is1, os0, os1, qs0, qs1,
            *, sd, chw, nch, nb, stages):
    cid = lax.axis_index("c")
    sid = lax.axis_index("s")
    wid = sid * _NC + cid
    base = wid * nch * chw

    inb = (in0, in1)
    oub = (ou0, ou1)
    psb = (ps0, ps1)
    ins = (is0, is1)
    oss = (os0, os1)
    pss = (qs0, qs1)

    def in_slice(t):
        i = t // nb
        bi = t % nb
        return in_hbm.at[pl.ds(bi * sd + base + i * chw, chw)]

    def out_slice(t):
        i = t // nb
        bi = t % nb
        return out_hbm.at[pl.ds(bi * sd + base + i * chw, chw)]

    def pos_slice(i):
        return pos_hbm.at[pl.ds(base + i * chw, chw)]

    pltpu.async_copy(pos_slice(0), ps0, qs0)
    pltpu.async_copy(pos_slice(1), ps1, qs1)
    pltpu.async_copy(in_slice(0), in0, is0)
    pltpu.async_copy(in_slice(1), in1, is1)

    def body(q, _):
        for k in range(2 * nb):
            p = k % 2
            pc = k // nb
            t = 2 * nb * q + k
            i_chunk = 2 * q + pc

            pltpu.make_async_copy(in_slice(t), inb[p], ins[p]).wait()
            if k % nb == 0:
                pltpu.make_async_copy(pos_slice(i_chunk), psb[pc], pss[pc]).wait()

            @pl.when(t >= 2)
            def _():
                pltpu.make_async_copy(oub[p], out_slice(t - 2), oss[p]).wait()

            @plsc.parallel_loop(0, chw, 16, unroll=8)
            def _add(j):
                oub[p][pl.ds(j, 16)] = (
                    inb[p][pl.ds(j, 16)] + psb[pc][pl.ds(j, 16)]
                )

            pltpu.async_copy(oub[p], out_slice(t), oss[p])

            @pl.when(t + 2 < stages)
            def _():
                pltpu.async_copy(in_slice(t + 2), inb[p], ins[p])

            if k % nb == nb - 1:
                nxt = i_chunk + 2

                @pl.when(nxt < nch)
                def _():
                    pltpu.async_copy(pos_slice(nxt), psb[pc], pss[pc])
        return 0

    lax.fori_loop(0, stages // (2 * nb), body, 0)

    for t in (stages - 2, stages - 1):
        p = t % 2
        pltpu.make_async_copy(oub[p], out_slice(t), oss[p]).wait()


def kernel(inputs, pos_table):
    B, S, D = inputs.shape
    CHW = _CH * D
    NCH = S // (_NW * _CH)
    SD = S * D
    STAGES = NCH * B

    mesh = plsc.VectorSubcoreMesh(core_axis_name="c", subcore_axis_name="s")
    body = functools.partial(_sc_add, sd=SD, chw=CHW, nch=NCH, nb=B,
                             stages=STAGES)

    out_flat = pl.kernel(
        body,
        out_type=jax.ShapeDtypeStruct((B * SD,), jnp.float32),
        mesh=mesh,
        scratch_types=[pltpu.VMEM((CHW,), jnp.float32)] * 6
        + [pltpu.SemaphoreType.DMA] * 6,
    )(inputs.reshape(B * SD), pos_table.reshape(SD))
    return out_flat.reshape(B, S, D)


# speedup vs baseline: 1.2453x; 1.0002x over previous
import functools

import jax
import jax.numpy as jnp
from jax import lax
from jax.experimental import pallas as pl
from jax.experimental.pallas import tpu as pltpu
from jax.experimental.pallas import tpu_sc as plsc

_NC = 2
_NS = 16
_NW = _NC * _NS

_CH = 8


def _sc_add(in_hbm, pos_hbm, out_hbm,
            in0, in1, ou0, ou1, ps0, ps1,
            is0, is1, os0, os1, qs0, qs1,
            *, sd, chw, nch, nb, stages):
    cid = lax.axis_index("c")
    sid = lax.axis_index("s")
    wid = sid * _NC + cid
    base = wid * nch * chw

    inb = (in0, in1)
    oub = (ou0, ou1)
    psb = (ps0, ps1)
    ins = (is0, is1)
    oss = (os0, os1)
    pss = (qs0, qs1)

    def in_slice(t):
        i = t // nb
        bi = t % nb
        return in_hbm.at[pl.ds(bi * sd + base + i * chw, chw)]

    def out_slice(t):
        i = t // nb
        bi = t % nb
        return out_hbm.at[pl.ds(bi * sd + base + i * chw, chw)]

    def pos_slice(i):
        return pos_hbm.at[pl.ds(base + i * chw, chw)]

    pltpu.async_copy(pos_slice(0), ps0, qs0)
    pltpu.async_copy(pos_slice(1), ps1, qs1)
    pltpu.async_copy(in_slice(0), in0, is0)
    pltpu.async_copy(in_slice(1), in1, is1)

    def body(q, _):
        for k in range(2 * nb):
            p = k % 2
            pc = k // nb
            t = 2 * nb * q + k
            i_chunk = 2 * q + pc

            pltpu.make_async_copy(in_slice(t), inb[p], ins[p]).wait()
            if k % nb == 0:
                pltpu.make_async_copy(pos_slice(i_chunk), psb[pc], pss[pc]).wait()

            @pl.when(t >= 2)
            def _():
                pltpu.make_async_copy(oub[p], out_slice(t - 2), oss[p]).wait()

            @plsc.parallel_loop(0, chw, 16, unroll=16)
            def _add(j):
                oub[p][pl.ds(j, 16)] = (
                    inb[p][pl.ds(j, 16)] + psb[pc][pl.ds(j, 16)]
                )

            pltpu.async_copy(oub[p], out_slice(t), oss[p])

            @pl.when(t + 2 < stages)
            def _():
                pltpu.async_copy(in_slice(t + 2), inb[p], ins[p])

            if k % nb == nb - 1:
                nxt = i_chunk + 2

                @pl.when(nxt < nch)
                def _():
                    pltpu.async_copy(pos_slice(nxt), psb[pc], pss[pc])
        return 0

    lax.fori_loop(0, stages // (2 * nb), body, 0)

    for t in (stages - 2, stages - 1):
        p = t % 2
        pltpu.make_async_copy(oub[p], out_slice(t), oss[p]).wait()


def kernel(inputs, pos_table):
    B, S, D = inputs.shape
    CHW = _CH * D
    NCH = S // (_NW * _CH)
    SD = S * D
    STAGES = NCH * B

    mesh = plsc.VectorSubcoreMesh(core_axis_name="c", subcore_axis_name="s")
    body = functools.partial(_sc_add, sd=SD, chw=CHW, nch=NCH, nb=B,
                             stages=STAGES)

    out_flat = pl.kernel(
        body,
        out_type=jax.ShapeDtypeStruct((B * SD,), jnp.float32),
        mesh=mesh,
        scratch_types=[pltpu.VMEM((CHW,), jnp.float32)] * 6
        + [pltpu.SemaphoreType.DMA] * 6,
    )(inputs.reshape(B * SD), pos_table.reshape(SD))
    return out_flat.reshape(B, S, D)


# speedup vs baseline: 1.3680x; 1.0985x over previous
import functools

import jax
import jax.numpy as jnp
from jax import lax
from jax.experimental import pallas as pl
from jax.experimental.pallas import tpu as pltpu
from jax.experimental.pallas import tpu_sc as plsc

_NC = 2
_NS = 16
_NW = _NC * _NS
_CH = 8
_S_SC = 2048


def _sc_add(in_hbm, pos_hbm, out_hbm,
            in0, in1, ou0, ou1, ps0, ps1,
            is0, is1, os0, os1, qs0, qs1,
            *, sd, sd_out, chw, nch, nb, stages):
    cid = lax.axis_index("c")
    sid = lax.axis_index("s")
    wid = sid * _NC + cid
    base = wid * nch * chw

    inb = (in0, in1)
    oub = (ou0, ou1)
    psb = (ps0, ps1)
    ins = (is0, is1)
    oss = (os0, os1)
    pss = (qs0, qs1)

    def in_slice(t):
        i = t // nb
        bi = t % nb
        return in_hbm.at[pl.ds(bi * sd + base + i * chw, chw)]

    def out_slice(t):
        i = t // nb
        bi = t % nb
        return out_hbm.at[pl.ds(bi * sd_out + base + i * chw, chw)]

    def pos_slice(i):
        return pos_hbm.at[pl.ds(base + i * chw, chw)]

    pltpu.async_copy(pos_slice(0), ps0, qs0)
    pltpu.async_copy(pos_slice(1), ps1, qs1)
    pltpu.async_copy(in_slice(0), in0, is0)
    pltpu.async_copy(in_slice(1), in1, is1)

    def body(q, _):
        for k in range(2 * nb):
            p = k % 2
            pc = k // nb
            t = 2 * nb * q + k
            i_chunk = 2 * q + pc

            pltpu.make_async_copy(in_slice(t), inb[p], ins[p]).wait()
            if k % nb == 0:
                pltpu.make_async_copy(pos_slice(i_chunk), psb[pc], pss[pc]).wait()

            @pl.when(t >= 2)
            def _():
                pltpu.make_async_copy(oub[p], out_slice(t - 2), oss[p]).wait()

            @plsc.parallel_loop(0, chw, 16, unroll=8)
            def _add(j):
                oub[p][pl.ds(j, 16)] = (
                    inb[p][pl.ds(j, 16)] + psb[pc][pl.ds(j, 16)]
                )

            pltpu.async_copy(oub[p], out_slice(t), oss[p])

            @pl.when(t + 2 < stages)
            def _():
                pltpu.async_copy(in_slice(t + 2), inb[p], ins[p])

            if k % nb == nb - 1:
                nxt = i_chunk + 2

                @pl.when(nxt < nch)
                def _():
                    pltpu.async_copy(pos_slice(nxt), psb[pc], pss[pc])
        return 0

    lax.fori_loop(0, stages // (2 * nb), body, 0)

    for t in (stages - 2, stages - 1):
        p = t % 2
        pltpu.make_async_copy(oub[p], out_slice(t), oss[p]).wait()


def _add_block(x_ref, p_ref, o_ref):
    o_ref[...] = x_ref[...] + p_ref[...]


def kernel(inputs, pos_table):
    B, S, D = inputs.shape
    SD = S * D
    S_TC = S - _S_SC

    CHW = _CH * D
    NCH = _S_SC // (_NW * _CH)
    STAGES = NCH * B
    mesh = plsc.VectorSubcoreMesh(core_axis_name="c", subcore_axis_name="s")
    body = functools.partial(_sc_add, sd=SD, sd_out=_S_SC * D, chw=CHW,
                             nch=NCH, nb=B, stages=STAGES)
    out_sc = pl.kernel(
        body,
        out_type=jax.ShapeDtypeStruct((B * _S_SC * D,), jnp.float32),
        mesh=mesh,
        scratch_types=[pltpu.VMEM((CHW,), jnp.float32)] * 6
        + [pltpu.SemaphoreType.DMA] * 6,
    )(inputs.reshape(B * SD), pos_table.reshape(SD))

    SB = 1024
    OFF = _S_SC // SB
    out_tc = pl.pallas_call(
        _add_block,
        grid=(S_TC // SB, B),
        in_specs=[
            pl.BlockSpec((1, SB, D), lambda s, b: (b, s + OFF, 0)),
            pl.BlockSpec((SB, D), lambda s, b: (s + OFF, 0)),
        ],
        out_specs=pl.BlockSpec((1, SB, D), lambda s, b: (b, s, 0)),
        out_shape=jax.ShapeDtypeStruct((B, S_TC, D), jnp.float32),
    )(inputs, pos_table)

    return jnp.concatenate(
        [out_sc.reshape(B, _S_SC, D), out_tc], axis=1)


# speedup vs baseline: 4.9927x; 3.6495x over previous
import jax
import jax.numpy as jnp
from jax.experimental import pallas as pl


def _add_block(x_ref, p_ref, o_ref):
    o_ref[...] = x_ref[...] + p_ref[...]


def kernel(inputs, pos_table):
    B, S, D = inputs.shape
    SB = 1024
    return pl.pallas_call(
        _add_block,
        grid=(S // SB, B),
        in_specs=[
            pl.BlockSpec((1, SB, D), lambda s, b: (b, s, 0)),
            pl.BlockSpec((SB, D), lambda s, b: (s, 0)),
        ],
        out_specs=pl.BlockSpec((1, SB, D), lambda s, b: (b, s, 0)),
        out_shape=jax.ShapeDtypeStruct(inputs.shape, inputs.dtype),
    )(inputs, pos_table)
